# Initial kernel scaffold; baseline (speedup 1.0000x reference)
import jax
import jax.numpy as jnp
from jax.experimental import pallas as pl


def kernel(query, reference_points, input_flatten, input_spatial_shapes, input_level_start_index, W_off, b_off, W_attn, b_attn, W_val, b_val, W_out, b_out):
    raise NotImplementedError("write your pallas kernel here")


# speedup vs baseline: 12.9031x; 12.9031x over previous
import functools
import numpy as np
import jax
import jax.numpy as jnp
from jax import lax
from jax.experimental import pallas as pl
from jax.experimental.pallas import tpu as pltpu
from jax.experimental.pallas import tpu_sc as plsc

B = 2
D = 256
M = 8
L = 4
P = 4
DH = 32
SHAPES = ((128, 128), (64, 64), (32, 32), (16, 16))
LEN = sum(h * w for h, w in SHAPES)
ROWS = B * LEN
T = 256
NT = ROWS // T
TPB = LEN // T
NWORK = 32
RPW = ROWS // NWORK

_lanes = np.arange(M * L * P)
_m_ln = _lanes // (L * P)
_l_ln = (_lanes % (L * P)) // P
_W_np = np.array([w for h, w in SHAPES], np.float32)
_H_np = np.array([h for h, w in SHAPES], np.float32)
_start_np = np.concatenate([[0], np.cumsum([h * w for h, w in SHAPES[:-1]])]).astype(np.int64)

_WLM = (_W_np[_l_ln].astype(np.int64) * M).astype(np.int32).reshape(1, 128)
_BASEI = (_start_np[_l_ln] * M + _m_ln).astype(np.int32).reshape(1, 128)
_WM1 = (_W_np[_l_ln] - 1.0).reshape(1, 128).astype(np.float32)
_HM1 = (_H_np[_l_ln] - 1.0).reshape(1, 128).astype(np.float32)

_SX = np.zeros((8, 128), np.float32)
_SY = np.zeros((8, 128), np.float32)
for _ln in range(128):
    _SX[2 * _l_ln[_ln] + 0, _ln] = _W_np[_l_ln[_ln]]
    _SY[2 * _l_ln[_ln] + 1, _ln] = _H_np[_l_ln[_ln]]

_BLK = (_lanes[:, None] // 16 == _lanes[None, :] // 16).astype(np.float32)


def _prep_body(q_ref, x_ref, rp_ref, wval_ref, bval_ref, woffx_ref, woffy_ref,
               cbx_ref, cby_ref, wattn_ref, battn_ref, sx_ref, sy_ref, blk_ref,
               wlm_ref, basei_ref, wm1_ref, hm1_ref,
               val_out, idx_out, w_out):
    q = q_ref[...]
    v = jnp.dot(x_ref[...], wval_ref[...], preferred_element_type=jnp.float32) + bval_ref[...]
    val_out[...] = v

    hi = jax.lax.Precision.HIGHEST
    gx = (jnp.dot(q, woffx_ref[...], preferred_element_type=jnp.float32, precision=hi)
          + jnp.dot(rp_ref[...], sx_ref[...], preferred_element_type=jnp.float32, precision=hi)
          + cbx_ref[...])
    gy = (jnp.dot(q, woffy_ref[...], preferred_element_type=jnp.float32, precision=hi)
          + jnp.dot(rp_ref[...], sy_ref[...], preferred_element_type=jnp.float32, precision=hi)
          + cby_ref[...])

    a = jnp.dot(q, wattn_ref[...], preferred_element_type=jnp.float32, precision=hi) + battn_ref[...]
    e = jnp.exp(a - jnp.max(a, axis=1, keepdims=True))
    aw = e / jnp.dot(e, blk_ref[...], preferred_element_type=jnp.float32)

    x0 = jnp.floor(gx)
    y0 = jnp.floor(gy)
    fx = gx - x0
    fy = gy - y0

    b = pl.program_id(0) // TPB
    boff = (b * (LEN * M)).astype(jnp.int32)
    wlm = wlm_ref[...]
    basei = basei_ref[...]
    wm1 = wm1_ref[...]
    hm1 = hm1_ref[...]

    idx_list = []
    w_list = []
    for dx, dy in ((0, 0), (1, 0), (0, 1), (1, 1)):
        xi = x0 + dx
        yi = y0 + dy
        valid = (xi >= 0.0) & (xi <= wm1) & (yi >= 0.0) & (yi <= hm1)
        wt = aw * (fx if dx else 1.0 - fx) * (fy if dy else 1.0 - fy)
        w_list.append(jnp.where(valid, wt, 0.0))
        idx = yi.astype(jnp.int32) * wlm + xi.astype(jnp.int32) * M + basei + boff
        idx_list.append(jnp.where(valid, idx, 0))
    idx_out[...] = jnp.concatenate(idx_list, axis=1)
    w_out[...] = jnp.concatenate(w_list, axis=1)


def _prep(q2, x2, rp2, W_val, bval2, W_offx, W_offy, cbx, cby, W_attn, battn2,
          sx, sy, blk, wlm, basei, wm1, hm1):
    row_spec = lambda cols: pl.BlockSpec((T, cols), lambda i: (i, 0))
    full_spec = lambda r, c: pl.BlockSpec((r, c), lambda i: (0, 0))
    return pl.pallas_call(
        _prep_body,
        grid=(NT,),
        in_specs=[
            row_spec(D), row_spec(D), row_spec(8),
            full_spec(D, D), full_spec(1, D),
            full_spec(D, 128), full_spec(D, 128),
            full_spec(1, 128), full_spec(1, 128),
            full_spec(D, 128), full_spec(1, 128),
            full_spec(8, 128), full_spec(8, 128), full_spec(128, 128),
            full_spec(1, 128), full_spec(1, 128), full_spec(1, 128), full_spec(1, 128),
        ],
        out_specs=[row_spec(D), row_spec(512), row_spec(512)],
        out_shape=[
            jax.ShapeDtypeStruct((ROWS, D), jnp.float32),
            jax.ShapeDtypeStruct((ROWS, 512), jnp.int32),
            jax.ShapeDtypeStruct((ROWS, 512), jnp.float32),
        ],
    )(q2, x2, rp2, W_val, bval2, W_offx, W_offy, cbx, cby, W_attn, battn2,
      sx, sy, blk, wlm, basei, wm1, hm1)


@functools.cache
def _sample_fn():
    mesh = plsc.VectorSubcoreMesh(core_axis_name="c", subcore_axis_name="s",
                                  num_cores=2, num_subcores=16)

    @functools.partial(
        pl.kernel,
        out_type=jax.ShapeDtypeStruct((ROWS, D), jnp.float32),
        mesh=mesh,
        scratch_types=[
            pltpu.VMEM((4, 128), jnp.int32),
            pltpu.VMEM((512,), jnp.float32),
            pltpu.VMEM((4, 128, DH), jnp.float32),
            pltpu.VMEM((D,), jnp.float32),
            pltpu.SemaphoreType.DMA,
        ],
        compiler_params=pltpu.CompilerParams(use_tc_tiling_on_sc=False),
    )
    def _sample(value_hbm, idx_hbm, w_hbm, out_hbm, idx_v, w_v, rows_v, out_v, sem):
        wid = lax.axis_index("s") * 2 + lax.axis_index("c")
        base = wid * RPW

        def body(r, carry):
            q = base + r
            pltpu.sync_copy(idx_hbm.at[pl.ds(q * 4, 4)], idx_v)
            pltpu.sync_copy(w_hbm.at[q], w_v)
            for t in range(4):
                pltpu.async_copy(value_hbm.at[idx_v.at[t]], rows_v.at[t], sem).wait()
            for m in range(M):
                acc0 = jnp.zeros((16,), jnp.float32)
                acc1 = jnp.zeros((16,), jnp.float32)
                for t in range(4):
                    wvec = w_v[pl.ds(t * 128 + m * 16, 16)]
                    for j in range(16):
                        ws = wvec[j]
                        acc0 = acc0 + ws * rows_v[t, m * 16 + j, 0:16]
                        acc1 = acc1 + ws * rows_v[t, m * 16 + j, 16:32]
                out_v[pl.ds(m * 32, 16)] = acc0
                out_v[pl.ds(m * 32 + 16, 16)] = acc1
            pltpu.sync_copy(out_v, out_hbm.at[q])
            return carry

        lax.fori_loop(0, RPW, body, 0)

    return _sample


def _proj_body(o_ref, w_ref, b_ref, out_ref):
    out_ref[...] = (jnp.dot(o_ref[...], w_ref[...], preferred_element_type=jnp.float32)
                    + b_ref[...])


def _proj(o2, W_out, bout2):
    return pl.pallas_call(
        _proj_body,
        grid=(NT,),
        in_specs=[
            pl.BlockSpec((T, D), lambda i: (i, 0)),
            pl.BlockSpec((D, D), lambda i: (0, 0)),
            pl.BlockSpec((1, D), lambda i: (0, 0)),
        ],
        out_specs=pl.BlockSpec((T, D), lambda i: (i, 0)),
        out_shape=jax.ShapeDtypeStruct((ROWS, D), jnp.float32),
    )(o2, W_out, bout2)


def kernel(query, reference_points, input_flatten, input_spatial_shapes,
           input_level_start_index, W_off, b_off, W_attn, b_attn, W_val, b_val,
           W_out, b_out):
    q2 = query.reshape(ROWS, D)
    x2 = input_flatten.reshape(ROWS, D)
    rp2 = reference_points.reshape(ROWS, L * 2)
    W_offx = W_off[:, 0::2]
    W_offy = W_off[:, 1::2]
    cbx = (b_off[0::2] - 0.5).reshape(1, 128)
    cby = (b_off[1::2] - 0.5).reshape(1, 128)
    battn2 = b_attn.reshape(1, 128)
    bval2 = b_val.reshape(1, D)

    val, idx_all, w_all = _prep(
        q2, x2, rp2, W_val, bval2, W_offx, W_offy, cbx, cby, W_attn, battn2,
        jnp.asarray(_SX), jnp.asarray(_SY), jnp.asarray(_BLK),
        jnp.asarray(_WLM), jnp.asarray(_BASEI), jnp.asarray(_WM1), jnp.asarray(_HM1))

    value2 = val.reshape(ROWS * M, DH)
    idx2 = idx_all.reshape(ROWS * 4, 128)
    out_mid = _sample_fn()(value2, idx2, w_all)
    out = _proj(out_mid, W_out, b_out.reshape(1, D))
    return out.reshape(B, LEN, D)


# speedup vs baseline: 12.9151x; 1.0009x over previous
import functools
import numpy as np
import jax
import jax.numpy as jnp
from jax import lax
from jax.experimental import pallas as pl
from jax.experimental.pallas import tpu as pltpu
from jax.experimental.pallas import tpu_sc as plsc

B = 2
D = 256
M = 8
L = 4
P = 4
DH = 32
SHAPES = ((128, 128), (64, 64), (32, 32), (16, 16))
LEN = sum(h * w for h, w in SHAPES)
ROWS = B * LEN
T = 256
NT = ROWS // T
TPB = LEN // T
NWORK = 32
RPW = ROWS // NWORK

_lanes = np.arange(M * L * P)
_m_ln = _lanes // (L * P)
_l_ln = (_lanes % (L * P)) // P
_W_np = np.array([w for h, w in SHAPES], np.float32)
_H_np = np.array([h for h, w in SHAPES], np.float32)
_start_np = np.concatenate([[0], np.cumsum([h * w for h, w in SHAPES[:-1]])]).astype(np.int64)

_WLM = (_W_np[_l_ln].astype(np.int64) * M).astype(np.int32).reshape(1, 128)
_BASEI = (_start_np[_l_ln] * M + _m_ln).astype(np.int32).reshape(1, 128)
_WM1 = (_W_np[_l_ln] - 1.0).reshape(1, 128).astype(np.float32)
_HM1 = (_H_np[_l_ln] - 1.0).reshape(1, 128).astype(np.float32)

_SX = np.zeros((8, 128), np.float32)
_SY = np.zeros((8, 128), np.float32)
for _ln in range(128):
    _SX[2 * _l_ln[_ln] + 0, _ln] = _W_np[_l_ln[_ln]]
    _SY[2 * _l_ln[_ln] + 1, _ln] = _H_np[_l_ln[_ln]]

_BLK = (_lanes[:, None] // 16 == _lanes[None, :] // 16).astype(np.float32)


def _prep_body(q_ref, x_ref, rp_ref, wval_ref, bval_ref, woffx_ref, woffy_ref,
               cbx_ref, cby_ref, wattn_ref, battn_ref, sx_ref, sy_ref, blk_ref,
               wlm_ref, basei_ref, wm1_ref, hm1_ref,
               val_out, idx_out, w_out):
    q = q_ref[...]
    v = jnp.dot(x_ref[...], wval_ref[...], preferred_element_type=jnp.float32) + bval_ref[...]
    val_out[...] = v

    hi = jax.lax.Precision.HIGHEST
    gx = (jnp.dot(q, woffx_ref[...], preferred_element_type=jnp.float32, precision=hi)
          + jnp.dot(rp_ref[...], sx_ref[...], preferred_element_type=jnp.float32, precision=hi)
          + cbx_ref[...])
    gy = (jnp.dot(q, woffy_ref[...], preferred_element_type=jnp.float32, precision=hi)
          + jnp.dot(rp_ref[...], sy_ref[...], preferred_element_type=jnp.float32, precision=hi)
          + cby_ref[...])

    a = jnp.dot(q, wattn_ref[...], preferred_element_type=jnp.float32, precision=hi) + battn_ref[...]
    e = jnp.exp(a - jnp.max(a, axis=1, keepdims=True))
    aw = e / jnp.dot(e, blk_ref[...], preferred_element_type=jnp.float32)

    x0 = jnp.floor(gx)
    y0 = jnp.floor(gy)
    fx = gx - x0
    fy = gy - y0

    b = pl.program_id(0) // TPB
    boff = (b * (LEN * M)).astype(jnp.int32)
    wlm = wlm_ref[...]
    basei = basei_ref[...]
    wm1 = wm1_ref[...]
    hm1 = hm1_ref[...]

    idx_list = []
    w_list = []
    for dx, dy in ((0, 0), (1, 0), (0, 1), (1, 1)):
        xi = x0 + dx
        yi = y0 + dy
        valid = (xi >= 0.0) & (xi <= wm1) & (yi >= 0.0) & (yi <= hm1)
        wt = aw * (fx if dx else 1.0 - fx) * (fy if dy else 1.0 - fy)
        w_list.append(jnp.where(valid, wt, 0.0))
        idx = yi.astype(jnp.int32) * wlm + xi.astype(jnp.int32) * M + basei + boff
        idx_list.append(jnp.where(valid, idx, 0))
    idx_out[...] = jnp.concatenate(idx_list, axis=1)
    w_out[...] = jnp.concatenate(w_list, axis=1)


def _prep(q2, x2, rp2, W_val, bval2, W_offx, W_offy, cbx, cby, W_attn, battn2,
          sx, sy, blk, wlm, basei, wm1, hm1):
    row_spec = lambda cols: pl.BlockSpec((T, cols), lambda i: (i, 0))
    full_spec = lambda r, c: pl.BlockSpec((r, c), lambda i: (0, 0))
    return pl.pallas_call(
        _prep_body,
        grid=(NT,),
        in_specs=[
            row_spec(D), row_spec(D), row_spec(8),
            full_spec(D, D), full_spec(1, D),
            full_spec(D, 128), full_spec(D, 128),
            full_spec(1, 128), full_spec(1, 128),
            full_spec(D, 128), full_spec(1, 128),
            full_spec(8, 128), full_spec(8, 128), full_spec(128, 128),
            full_spec(1, 128), full_spec(1, 128), full_spec(1, 128), full_spec(1, 128),
        ],
        out_specs=[row_spec(D), row_spec(512), row_spec(512)],
        out_shape=[
            jax.ShapeDtypeStruct((ROWS, D), jnp.float32),
            jax.ShapeDtypeStruct((ROWS, 512), jnp.int32),
            jax.ShapeDtypeStruct((ROWS, 512), jnp.float32),
        ],
    )(q2, x2, rp2, W_val, bval2, W_offx, W_offy, cbx, cby, W_attn, battn2,
      sx, sy, blk, wlm, basei, wm1, hm1)


QC = 20
QC4 = QC * 4
NCH = RPW // QC


@functools.cache
def _sample_fn():
    mesh = plsc.VectorSubcoreMesh(core_axis_name="c", subcore_axis_name="s",
                                  num_cores=2, num_subcores=16)

    @functools.partial(
        pl.kernel,
        out_type=jax.ShapeDtypeStruct((ROWS, D), jnp.float32),
        mesh=mesh,
        scratch_types=[
            pltpu.VMEM((2, QC4, 128), jnp.int32),
            pltpu.VMEM((2, QC, 512), jnp.float32),
            pltpu.VMEM((2, 4, 128, DH), jnp.float32),
            pltpu.VMEM((QC, D), jnp.float32),
            pltpu.SemaphoreType.DMA,
            pltpu.SemaphoreType.DMA,
            pltpu.SemaphoreType.DMA,
        ],
        compiler_params=pltpu.CompilerParams(use_tc_tiling_on_sc=False),
    )
    def _sample(value_hbm, idx_hbm, w_hbm, out_hbm,
                idxc, wc, rowb, outc, sem_ch, sem_g0, sem_g1):
        wid = lax.axis_index("s") * 2 + lax.axis_index("c")
        base = wid * RPW

        def fire_chunk(c):
            q0 = jnp.minimum(base + c * QC, ROWS - QC)
            s = lax.rem(c, 2)
            pltpu.async_copy(idx_hbm.at[pl.ds(q0 * 4, QC4)], idxc.at[s], sem_ch)
            pltpu.async_copy(w_hbm.at[pl.ds(q0, QC)], wc.at[s], sem_ch)

        def wait_chunk():
            pltpu.make_async_copy(idx_hbm.at[pl.ds(0, QC4)], idxc.at[0], sem_ch).wait()
            pltpu.make_async_copy(w_hbm.at[pl.ds(0, QC)], wc.at[0], sem_ch).wait()

        def fire_row(s, r, rb, sem):
            for t in range(4):
                pltpu.async_copy(value_hbm.at[idxc.at[s, r * 4 + t]],
                                 rowb.at[rb, t], sem)

        def wait_row(rb, sem):
            for t in range(4):
                pltpu.make_async_copy(value_hbm.at[pl.ds(0, 128)],
                                      rowb.at[rb, t], sem).wait()

        def accum_row(s, r, rb):
            def mbody(m, carry):
                acc = [jnp.zeros((16,), jnp.float32) for _ in range(2)]
                for t in range(4):
                    wvec = wc[s, r, pl.ds(t * 128 + m * 16, 16)]
                    for j in range(16):
                        ws = wvec[j]
                        acc[0] = acc[0] + ws * rowb[rb, t, m * 16 + j, 0:16]
                        acc[1] = acc[1] + ws * rowb[rb, t, m * 16 + j, 16:32]
                outc[r, pl.ds(m * 32, 16)] = acc[0]
                outc[r, pl.ds(m * 32 + 16, 16)] = acc[1]
                return carry
            lax.fori_loop(0, M, mbody, 0)

        def chunk_body(c, carry):
            s = lax.rem(c, 2)
            def pair_body(p, carry2):
                fire_row(s, 2 * p + 1, 1, sem_g1)
                wait_row(0, sem_g0)
                accum_row(s, 2 * p, 0)

                @pl.when(2 * p + 2 < QC)
                def _():
                    fire_row(s, 2 * p + 2, 0, sem_g0)
                wait_row(1, sem_g1)
                accum_row(s, 2 * p + 1, 1)
                return carry2
            lax.fori_loop(0, QC // 2, pair_body, 0)
            pltpu.sync_copy(outc, out_hbm.at[pl.ds(base + c * QC, QC)])
            wait_chunk()
            fire_chunk(c + 2)
            fire_row(1 - s, 0, 0, sem_g0)
            return carry

        fire_chunk(jnp.int32(0))
        wait_chunk()
        fire_chunk(jnp.int32(1))
        fire_row(jnp.int32(0), jnp.int32(0), 0, sem_g0)
        lax.fori_loop(0, NCH, chunk_body, 0)
        wait_row(0, sem_g0)
        wait_chunk()

    return _sample


def _proj_body(o_ref, w_ref, b_ref, out_ref):
    out_ref[...] = (jnp.dot(o_ref[...], w_ref[...], preferred_element_type=jnp.float32)
                    + b_ref[...])


def _proj(o2, W_out, bout2):
    return pl.pallas_call(
        _proj_body,
        grid=(NT,),
        in_specs=[
            pl.BlockSpec((T, D), lambda i: (i, 0)),
            pl.BlockSpec((D, D), lambda i: (0, 0)),
            pl.BlockSpec((1, D), lambda i: (0, 0)),
        ],
        out_specs=pl.BlockSpec((T, D), lambda i: (i, 0)),
        out_shape=jax.ShapeDtypeStruct((ROWS, D), jnp.float32),
    )(o2, W_out, bout2)


def kernel(query, reference_points, input_flatten, input_spatial_shapes,
           input_level_start_index, W_off, b_off, W_attn, b_attn, W_val, b_val,
           W_out, b_out):
    q2 = query.reshape(ROWS, D)
    x2 = input_flatten.reshape(ROWS, D)
    rp2 = reference_points.reshape(ROWS, L * 2)
    W_offx = W_off[:, 0::2]
    W_offy = W_off[:, 1::2]
    cbx = (b_off[0::2] - 0.5).reshape(1, 128)
    cby = (b_off[1::2] - 0.5).reshape(1, 128)
    battn2 = b_attn.reshape(1, 128)
    bval2 = b_val.reshape(1, D)

    val, idx_all, w_all = _prep(
        q2, x2, rp2, W_val, bval2, W_offx, W_offy, cbx, cby, W_attn, battn2,
        jnp.asarray(_SX), jnp.asarray(_SY), jnp.asarray(_BLK),
        jnp.asarray(_WLM), jnp.asarray(_BASEI), jnp.asarray(_WM1), jnp.asarray(_HM1))

    value2 = val.reshape(ROWS * M, DH)
    idx2 = idx_all.reshape(ROWS * 4, 128)
    out_mid = _sample_fn()(value2, idx2, w_all)
    out = _proj(out_mid, W_out, b_out.reshape(1, D))
    return out.reshape(B, LEN, D)
